# baseline (speedup 1.0000x reference)
import functools

import jax
import jax.numpy as jnp
from jax import lax
from jax.experimental import pallas as pl
from jax.experimental.pallas import tpu as pltpu
from jax.experimental.pallas import tpu_sc as plsc

_K = 8192
_D = 256
_M = 256
_CHUNK_C = 2736
_NCHUNK = 3
_KPAD = _CHUNK_C * _NCHUNK
_NC, _NS = 2, 16
_NW = _NC * _NS
_GCHUNK = 128


def _argmin_body(x_ref, emb_ref, xsq_ref, idx_ref, acc_v_all, acc_i_all):
    c = pl.program_id(0)
    t = pl.program_id(1)
    acc_v = acc_v_all.at[pl.ds(t * _M, _M), :]
    acc_i = acc_i_all.at[pl.ds(t * _M, _M), :]
    x = x_ref[...]
    xsq = xsq_ref[0]
    emb = emb_ref[...]
    xe = lax.dot_general(x, emb, (((1,), (1,)), ((), ())),
                         preferred_element_type=jnp.float32)
    d = xsq - 2.0 * xe
    m = jnp.min(d, axis=1, keepdims=True)
    ids = lax.broadcasted_iota(jnp.int32, d.shape, 1)
    bi = jnp.min(jnp.where(d == m, ids, jnp.int32(2**30)),
                 axis=1, keepdims=True) + c * _CHUNK_C

    @pl.when(c == 0)
    def _():
        acc_i[...] = bi
        acc_v[...] = m.astype(jnp.bfloat16).astype(jnp.float32)

    @pl.when(c > 0)
    def _():
        win = m < acc_v[...]
        acc_i[...] = jnp.where(win, bi, acc_i[...])
        acc_v[...] = jnp.where(win, m, acc_v[...]).astype(
            jnp.bfloat16).astype(jnp.float32)

    @pl.when(c == _NCHUNK - 1)
    def _():
        idx_ref[0, 0, :] = acc_i[...][:, 0]


def _nearest_indices(x_flat, x_sq, emb_pad):
    t_blocks = x_flat.shape[0] // _M
    out = pl.pallas_call(
        _argmin_body,
        grid=(_NCHUNK, t_blocks),
        in_specs=[
            pl.BlockSpec((_M, _D), lambda c, t: (t, 0)),
            pl.BlockSpec((_CHUNK_C, _D), lambda c, t: (c, 0)),
            pl.BlockSpec((1, _M, 1), lambda c, t: (t, 0, 0)),
        ],
        out_specs=pl.BlockSpec((1, 1, _M), lambda c, t: (t, 0, 0)),
        out_shape=jax.ShapeDtypeStruct((t_blocks, 1, _M), jnp.int32),
        scratch_shapes=[
            pltpu.VMEM((t_blocks * _M, 1), jnp.float32),
            pltpu.VMEM((t_blocks * _M, 1), jnp.int32),
        ],
    )(x_flat, emb_pad, x_sq.reshape(t_blocks, _M, 1))
    return out.reshape(-1)


def _gather_body(idx_hbm, table_hbm, out_hbm, idx_v, rows_v, sem):
    wid = lax.axis_index("s") * _NC + lax.axis_index("c")
    n_chunks = idx_hbm.shape[1]
    pltpu.sync_copy(idx_hbm.at[wid], idx_v)
    for j in range(n_chunks):
        pltpu.async_copy(table_hbm.at[idx_v.at[j]], rows_v, sem).wait()
        base = wid * (n_chunks * _GCHUNK) + j * _GCHUNK
        pltpu.sync_copy(rows_v, out_hbm.at[pl.ds(base, _GCHUNK)])


def _gather_rows(indices, emb_weight):
    b = indices.shape[0]
    n_chunks = b // (_NW * _GCHUNK)
    idx3 = indices.reshape(_NW, n_chunks, _GCHUNK)
    mesh = plsc.VectorSubcoreMesh(core_axis_name="c", subcore_axis_name="s")
    kern = pl.kernel(
        _gather_body,
        out_type=jax.ShapeDtypeStruct((b, _D), jnp.float32),
        mesh=mesh,
        scratch_types=[
            pltpu.VMEM((n_chunks, _GCHUNK), jnp.int32),
            pltpu.VMEM((_GCHUNK, _D), jnp.float32),
            pltpu.SemaphoreType.DMA,
        ],
    )
    return kern(idx3, emb_weight)


def kernel(x, emb_weight):
    ndim = x.ndim
    new_dims = (0,) + tuple(range(2, ndim)) + (1,)
    x_permuted = jnp.transpose(x, new_dims)
    permuted_shape = x_permuted.shape
    x_flat = x_permuted.reshape(-1, permuted_shape[-1])
    x_sq = jnp.sum(x * x, axis=1).reshape(-1)
    emb_pad = jnp.concatenate(
        [emb_weight, jnp.zeros((_KPAD - _K, _D), jnp.float32)], axis=0)

    indices = _nearest_indices(x_flat, x_sq, emb_pad)
    quantised_flat = _gather_rows(indices, emb_weight)

    quantised_permuted = quantised_flat.reshape(permuted_shape)
    old_dims = (0,) + (ndim - 1,) + tuple(range(1, ndim - 1))
    return jnp.transpose(quantised_permuted, old_dims)


# speedup vs baseline: 1.0145x; 1.0145x over previous
import functools

import jax
import jax.numpy as jnp
from jax import lax
from jax.experimental import pallas as pl
from jax.experimental.pallas import tpu as pltpu
from jax.experimental.pallas import tpu_sc as plsc

_K = 8192
_D = 256
_M = 256
_CHUNK_C = 2736
_NCHUNK = 3
_KPAD = _CHUNK_C * _NCHUNK
_NC, _NS = 2, 16
_NW = _NC * _NS
_GCHUNK = 128


def _argmin_body(x_ref, emb_ref, xsq_ref, idx_ref, acc_v_all, acc_i_all):
    c = pl.program_id(0)
    t = pl.program_id(1)
    acc_v = acc_v_all.at[pl.ds(t * _M, _M), :]
    acc_i = acc_i_all.at[pl.ds(t * _M, _M), :]
    x = x_ref[...]
    xsq = xsq_ref[0]
    half = _CHUNK_C // 2
    ms, bis = [], []
    for h in range(2):
        emb_h = emb_ref[pl.ds(h * half, half), :]
        xe = lax.dot_general(x, emb_h, (((1,), (1,)), ((), ())),
                             preferred_element_type=jnp.float32)
        d = xsq - 2.0 * xe
        m_h = jnp.min(d, axis=1, keepdims=True)
        ids = lax.broadcasted_iota(jnp.int32, d.shape, 1)
        i_h = jnp.min(jnp.where(d == m_h, ids, jnp.int32(2**30)),
                      axis=1, keepdims=True)
        ms.append(m_h)
        bis.append(i_h + (c * _CHUNK_C + h * half))
    first = ms[0] <= ms[1]
    m = jnp.where(first, ms[0], ms[1])
    bi = jnp.where(first, bis[0], bis[1])

    @pl.when(c == 0)
    def _():
        acc_i[...] = bi
        acc_v[...] = m.astype(jnp.bfloat16).astype(jnp.float32)

    @pl.when(c > 0)
    def _():
        win = m < acc_v[...]
        acc_i[...] = jnp.where(win, bi, acc_i[...])
        acc_v[...] = jnp.where(win, m, acc_v[...]).astype(
            jnp.bfloat16).astype(jnp.float32)

    @pl.when(c == _NCHUNK - 1)
    def _():
        idx_ref[0, 0, :] = acc_i[...][:, 0]


def _nearest_indices(x_flat, x_sq, emb_pad):
    t_blocks = x_flat.shape[0] // _M
    out = pl.pallas_call(
        _argmin_body,
        grid=(_NCHUNK, t_blocks),
        in_specs=[
            pl.BlockSpec((_M, _D), lambda c, t: (t, 0)),
            pl.BlockSpec((_CHUNK_C, _D), lambda c, t: (c, 0)),
            pl.BlockSpec((1, _M, 1), lambda c, t: (t, 0, 0)),
        ],
        out_specs=pl.BlockSpec((1, 1, _M), lambda c, t: (t, 0, 0)),
        out_shape=jax.ShapeDtypeStruct((t_blocks, 1, _M), jnp.int32),
        scratch_shapes=[
            pltpu.VMEM((t_blocks * _M, 1), jnp.float32),
            pltpu.VMEM((t_blocks * _M, 1), jnp.int32),
        ],
    )(x_flat, emb_pad, x_sq.reshape(t_blocks, _M, 1))
    return out.reshape(-1)


def _gather_body(idx_hbm, table_hbm, out_hbm, idx_v, rows_v, sem):
    wid = lax.axis_index("s") * _NC + lax.axis_index("c")
    n_chunks = idx_hbm.shape[1]
    pltpu.sync_copy(idx_hbm.at[wid], idx_v)
    for j in range(n_chunks):
        pltpu.async_copy(table_hbm.at[idx_v.at[j]], rows_v, sem).wait()
        base = wid * (n_chunks * _GCHUNK) + j * _GCHUNK
        pltpu.sync_copy(rows_v, out_hbm.at[pl.ds(base, _GCHUNK)])


def _gather_rows(indices, emb_weight):
    b = indices.shape[0]
    n_chunks = b // (_NW * _GCHUNK)
    idx3 = indices.reshape(_NW, n_chunks, _GCHUNK)
    mesh = plsc.VectorSubcoreMesh(core_axis_name="c", subcore_axis_name="s")
    kern = pl.kernel(
        _gather_body,
        out_type=jax.ShapeDtypeStruct((b, _D), jnp.float32),
        mesh=mesh,
        scratch_types=[
            pltpu.VMEM((n_chunks, _GCHUNK), jnp.int32),
            pltpu.VMEM((_GCHUNK, _D), jnp.float32),
            pltpu.SemaphoreType.DMA,
        ],
    )
    return kern(idx3, emb_weight)


def kernel(x, emb_weight):
    ndim = x.ndim
    new_dims = (0,) + tuple(range(2, ndim)) + (1,)
    x_permuted = jnp.transpose(x, new_dims)
    permuted_shape = x_permuted.shape
    x_flat = x_permuted.reshape(-1, permuted_shape[-1])
    x_sq = jnp.sum(x * x, axis=1).reshape(-1)
    emb_pad = jnp.concatenate(
        [emb_weight, jnp.zeros((_KPAD - _K, _D), jnp.float32)], axis=0)

    indices = _nearest_indices(x_flat, x_sq, emb_pad)
    quantised_flat = _gather_rows(indices, emb_weight)

    quantised_permuted = quantised_flat.reshape(permuted_shape)
    old_dims = (0,) + (ndim - 1,) + tuple(range(1, ndim - 1))
    return jnp.transpose(quantised_permuted, old_dims)
